# baseline (device time: 345271 ns/iter reference)
import jax
import jax.numpy as jnp
from jax import lax
from jax.experimental import pallas as pl
from jax.experimental.pallas import tpu as pltpu

N_DEV = 4
K_TAPS = 4


def kernel(x, k, Wp):
    B, S, C = x.shape
    C_out = Wp.shape[1]

    def compute_body(k_ref, wp_ref, x_ref, out_ref):
        xa = x_ref[0]
        kv = k_ref[...]
        pad = jnp.concatenate(
            [jnp.zeros((K_TAPS - 1, C), xa.dtype), xa], axis=0
        )
        acc = pad[0:S] * kv[0]
        for t in range(1, K_TAPS):
            acc = acc + pad[t:t + S] * kv[t]
        a = acc * jax.nn.sigmoid(acc)
        out_ref[0] = jnp.dot(
            a.astype(jnp.bfloat16),
            wp_ref[...].astype(jnp.bfloat16),
            preferred_element_type=jnp.float32,
        ).astype(jnp.bfloat16)

    partial = pl.pallas_call(
        compute_body,
        grid=(B,),
        in_specs=[
            pl.BlockSpec((K_TAPS, C), lambda b: (0, 0)),
            pl.BlockSpec((C, C_out), lambda b: (0, 0)),
            pl.BlockSpec((1, S, C), lambda b: (b, 0, 0)),
        ],
        out_specs=pl.BlockSpec((1, S, C_out), lambda b: (b, 0, 0)),
        out_shape=jax.ShapeDtypeStruct((B, S, C_out), jnp.bfloat16),
    )(k, Wp, x)

    def ar_body(p_ref, out_ref, comm_ref, rs_send, rs_recv, ag_send, ag_recv):
        my = lax.axis_index("i")
        left = lax.rem(my + N_DEV - 1, N_DEV)
        right = lax.rem(my + 1, N_DEV)

        barrier = pltpu.get_barrier_semaphore()
        for nbr in (left, right):
            pl.semaphore_signal(
                barrier, inc=1,
                device_id=(nbr,), device_id_type=pl.DeviceIdType.MESH,
            )
        pl.semaphore_wait(barrier, 2)

        out_ref[...] = p_ref[...]

        for t in range(N_DEV - 1):
            c_send = lax.rem(my + N_DEV - t, N_DEV)
            c_recv = lax.rem(my + N_DEV - t - 1, N_DEV)
            rdma = pltpu.make_async_remote_copy(
                src_ref=out_ref.at[pl.ds(c_send, 1)],
                dst_ref=comm_ref.at[pl.ds(t, 1)],
                send_sem=rs_send.at[t],
                recv_sem=rs_recv.at[t],
                device_id=(right,),
                device_id_type=pl.DeviceIdType.MESH,
            )
            rdma.start()
            rdma.wait()
            out_ref[pl.ds(c_recv, 1)] = (
                out_ref[pl.ds(c_recv, 1)] + comm_ref[pl.ds(t, 1)]
            )

        for t in range(N_DEV - 1):
            c_send = lax.rem(my + 1 + N_DEV - t, N_DEV)
            rdma = pltpu.make_async_remote_copy(
                src_ref=out_ref.at[pl.ds(c_send, 1)],
                dst_ref=out_ref.at[pl.ds(c_send, 1)],
                send_sem=ag_send.at[t],
                recv_sem=ag_recv.at[t],
                device_id=(right,),
                device_id_type=pl.DeviceIdType.MESH,
            )
            rdma.start()
            rdma.wait()

    out = pl.pallas_call(
        ar_body,
        out_shape=jax.ShapeDtypeStruct((B, S, C_out), jnp.bfloat16),
        in_specs=[pl.BlockSpec(memory_space=pltpu.VMEM)],
        out_specs=pl.BlockSpec(memory_space=pltpu.VMEM),
        scratch_shapes=[
            pltpu.VMEM((N_DEV - 1, S, C_out), jnp.bfloat16),
            pltpu.SemaphoreType.DMA((N_DEV - 1,)),
            pltpu.SemaphoreType.DMA((N_DEV - 1,)),
            pltpu.SemaphoreType.DMA((N_DEV - 1,)),
            pltpu.SemaphoreType.DMA((N_DEV - 1,)),
        ],
        compiler_params=pltpu.CompilerParams(collective_id=0),
    )(partial)
    return out


# device time: 210771 ns/iter; 1.6381x vs baseline; 1.6381x over previous
import jax
import jax.numpy as jnp
from jax import lax
from jax.experimental import pallas as pl
from jax.experimental.pallas import tpu as pltpu

N_DEV = 4
K_TAPS = 4


def kernel(x, k, Wp):
    B, S, C = x.shape
    C_out = Wp.shape[1]

    def compute_body(k_ref, wp_ref, x_ref, out_ref):
        xa = x_ref[0]
        kv = k_ref[...]
        pad = jnp.concatenate(
            [jnp.zeros((K_TAPS - 1, C), xa.dtype), xa], axis=0
        )
        acc = pad[0:S] * kv[0]
        for t in range(1, K_TAPS):
            acc = acc + pad[t:t + S] * kv[t]
        a = acc * jax.nn.sigmoid(acc)
        out_ref[0] = jnp.dot(
            a.astype(jnp.bfloat16),
            wp_ref[...].astype(jnp.bfloat16),
            preferred_element_type=jnp.float32,
        ).astype(jnp.bfloat16)

    partial = pl.pallas_call(
        compute_body,
        grid=(B,),
        in_specs=[
            pl.BlockSpec((K_TAPS, C), lambda b: (0, 0)),
            pl.BlockSpec((C, C_out), lambda b: (0, 0)),
            pl.BlockSpec((1, S, C), lambda b: (b, 0, 0)),
        ],
        out_specs=pl.BlockSpec((1, S, C_out), lambda b: (b, 0, 0)),
        out_shape=jax.ShapeDtypeStruct((B, S, C_out), jnp.bfloat16),
    )(k, Wp, x)

    H = C_out // 2

    def ar_body(p_ref, out_ref, comm_cw, comm_ccw,
                rs_send, rs_recv, ag_send, ag_recv):
        my = lax.axis_index("i")
        left = lax.rem(my + N_DEV - 1, N_DEV)
        right = lax.rem(my + 1, N_DEV)

        barrier = pltpu.get_barrier_semaphore()
        for nbr in (left, right):
            pl.semaphore_signal(
                barrier, inc=1,
                device_id=(nbr,), device_id_type=pl.DeviceIdType.MESH,
            )
        pl.semaphore_wait(barrier, 2)

        out_ref[...] = p_ref[...]

        for t in range(N_DEV - 1):
            cw_s = lax.rem(my + N_DEV - t, N_DEV)
            cw_r = lax.rem(my + N_DEV - t - 1, N_DEV)
            ccw_s = lax.rem(my + t, N_DEV)
            ccw_r = lax.rem(my + t + 1, N_DEV)
            r_cw = pltpu.make_async_remote_copy(
                src_ref=out_ref.at[pl.ds(cw_s, 1), :, pl.ds(0, H)],
                dst_ref=comm_cw.at[pl.ds(t, 1)],
                send_sem=rs_send.at[0, t],
                recv_sem=rs_recv.at[0, t],
                device_id=(right,),
                device_id_type=pl.DeviceIdType.MESH,
            )
            r_ccw = pltpu.make_async_remote_copy(
                src_ref=out_ref.at[pl.ds(ccw_s, 1), :, pl.ds(H, H)],
                dst_ref=comm_ccw.at[pl.ds(t, 1)],
                send_sem=rs_send.at[1, t],
                recv_sem=rs_recv.at[1, t],
                device_id=(left,),
                device_id_type=pl.DeviceIdType.MESH,
            )
            r_cw.start()
            r_ccw.start()
            r_cw.wait()
            r_ccw.wait()
            out_ref[pl.ds(cw_r, 1), :, pl.ds(0, H)] = (
                out_ref[pl.ds(cw_r, 1), :, pl.ds(0, H)]
                + comm_cw[pl.ds(t, 1)]
            )
            out_ref[pl.ds(ccw_r, 1), :, pl.ds(H, H)] = (
                out_ref[pl.ds(ccw_r, 1), :, pl.ds(H, H)]
                + comm_ccw[pl.ds(t, 1)]
            )

        for t in range(N_DEV - 1):
            cw_s = lax.rem(my + 1 + N_DEV - t, N_DEV)
            ccw_s = lax.rem(my + N_DEV - 1 + t, N_DEV)
            g_cw = pltpu.make_async_remote_copy(
                src_ref=out_ref.at[pl.ds(cw_s, 1), :, pl.ds(0, H)],
                dst_ref=out_ref.at[pl.ds(cw_s, 1), :, pl.ds(0, H)],
                send_sem=ag_send.at[0, t],
                recv_sem=ag_recv.at[0, t],
                device_id=(right,),
                device_id_type=pl.DeviceIdType.MESH,
            )
            g_ccw = pltpu.make_async_remote_copy(
                src_ref=out_ref.at[pl.ds(ccw_s, 1), :, pl.ds(H, H)],
                dst_ref=out_ref.at[pl.ds(ccw_s, 1), :, pl.ds(H, H)],
                send_sem=ag_send.at[1, t],
                recv_sem=ag_recv.at[1, t],
                device_id=(left,),
                device_id_type=pl.DeviceIdType.MESH,
            )
            g_cw.start()
            g_ccw.start()
            g_cw.wait()
            g_ccw.wait()

    out = pl.pallas_call(
        ar_body,
        out_shape=jax.ShapeDtypeStruct((B, S, C_out), jnp.bfloat16),
        in_specs=[pl.BlockSpec(memory_space=pltpu.VMEM)],
        out_specs=pl.BlockSpec(memory_space=pltpu.VMEM),
        scratch_shapes=[
            pltpu.VMEM((N_DEV - 1, S, H), jnp.bfloat16),
            pltpu.VMEM((N_DEV - 1, S, H), jnp.bfloat16),
            pltpu.SemaphoreType.DMA((2, N_DEV - 1)),
            pltpu.SemaphoreType.DMA((2, N_DEV - 1)),
            pltpu.SemaphoreType.DMA((2, N_DEV - 1)),
            pltpu.SemaphoreType.DMA((2, N_DEV - 1)),
        ],
        compiler_params=pltpu.CompilerParams(collective_id=0),
    )(partial)
    return out


# device time: 196703 ns/iter; 1.7553x vs baseline; 1.0715x over previous
import jax
import jax.numpy as jnp
from jax import lax
from jax.experimental import pallas as pl
from jax.experimental.pallas import tpu as pltpu

N_DEV = 4
K_TAPS = 4


def kernel(x, k, Wp):
    B, S, C = x.shape
    C_out = Wp.shape[1]
    H = C_out // 2

    def body(k_ref, wp_ref, x_ref, out_ref,
             xbuf, comm_cw, comm_ccw,
             rs_send, rs_recv, ag_send, ag_recv, ld_sems):
        my = lax.axis_index("i")
        left = lax.rem(my + N_DEV - 1, N_DEV)
        right = lax.rem(my + 1, N_DEV)
        order = [my, left, right, lax.rem(my + 2, N_DEV)]

        barrier = pltpu.get_barrier_semaphore()
        for nbr in (left, right):
            pl.semaphore_signal(
                barrier, inc=1,
                device_id=(nbr,), device_id_type=pl.DeviceIdType.MESH,
            )

        wp_bf = wp_ref[...].astype(jnp.bfloat16)
        kv = k_ref[...]

        def start_load(i):
            cp = pltpu.make_async_copy(
                x_ref.at[pl.ds(order[i], 1)],
                xbuf.at[pl.ds(i % 2, 1)],
                ld_sems.at[i % 2],
            )
            cp.start()
            return cp

        loads = {0: start_load(0), 1: start_load(1)}

        def compute(i):
            loads[i].wait()
            xa = xbuf[i % 2]
            pad = jnp.concatenate(
                [jnp.zeros((K_TAPS - 1, C), xa.dtype), xa], axis=0
            )
            acc = pad[0:S] * kv[0]
            for t in range(1, K_TAPS):
                acc = acc + pad[t:t + S] * kv[t]
            a = acc * jax.nn.sigmoid(acc)
            res = jnp.dot(
                a.astype(jnp.bfloat16), wp_bf,
                preferred_element_type=jnp.float32,
            ).astype(jnp.bfloat16)
            out_ref[pl.ds(order[i], 1)] = res[None]

        def rs_copy(dir_idx, t, c_send, comm, nbr):
            return pltpu.make_async_remote_copy(
                src_ref=out_ref.at[pl.ds(c_send, 1), :,
                                   pl.ds(dir_idx * H, H)],
                dst_ref=comm.at[pl.ds(t, 1)],
                send_sem=rs_send.at[dir_idx, t],
                recv_sem=rs_recv.at[dir_idx, t],
                device_id=(nbr,),
                device_id_type=pl.DeviceIdType.MESH,
            )

        def rs_add(dir_idx, t, c_recv, comm):
            sl = (pl.ds(c_recv, 1), slice(None), pl.ds(dir_idx * H, H))
            out_ref[sl] = out_ref[sl] + comm[pl.ds(t, 1)]

        def ag_copy(dir_idx, t, c_send, nbr):
            sl = (pl.ds(c_send, 1), slice(None), pl.ds(dir_idx * H, H))
            return pltpu.make_async_remote_copy(
                src_ref=out_ref.at[sl],
                dst_ref=out_ref.at[sl],
                send_sem=ag_send.at[dir_idx, t],
                recv_sem=ag_recv.at[dir_idx, t],
                device_id=(nbr,),
                device_id_type=pl.DeviceIdType.MESH,
            )

        compute(0)
        pl.semaphore_wait(barrier, 2)
        cw0 = rs_copy(0, 0, order[0], comm_cw, right)
        ccw0 = rs_copy(1, 0, order[0], comm_ccw, left)
        cw0.start()
        ccw0.start()
        loads[2] = start_load(2)
        compute(1)
        cw0.wait()
        rs_add(0, 0, order[1], comm_cw)
        cw1 = rs_copy(0, 1, order[1], comm_cw, right)
        cw1.start()
        loads[3] = start_load(3)
        compute(2)
        ccw0.wait()
        rs_add(1, 0, order[2], comm_ccw)
        ccw1 = rs_copy(1, 1, order[2], comm_ccw, left)
        ccw1.start()
        compute(3)
        cw1.wait()
        rs_add(0, 1, order[3], comm_cw)
        cw2 = rs_copy(0, 2, order[3], comm_cw, right)
        cw2.start()
        ccw1.wait()
        rs_add(1, 1, order[3], comm_ccw)
        ccw2 = rs_copy(1, 2, order[3], comm_ccw, left)
        ccw2.start()
        cw2.wait()
        rs_add(0, 2, order[2], comm_cw)
        ccw2.wait()
        rs_add(1, 2, order[1], comm_ccw)

        ag_cw_chunks = [order[2], order[0], order[1]]
        ag_ccw_chunks = [order[1], order[0], order[2]]
        for t in range(N_DEV - 1):
            g_cw = ag_copy(0, t, ag_cw_chunks[t], right)
            g_ccw = ag_copy(1, t, ag_ccw_chunks[t], left)
            g_cw.start()
            g_ccw.start()
            g_cw.wait()
            g_ccw.wait()

    out = pl.pallas_call(
        body,
        out_shape=jax.ShapeDtypeStruct((B, S, C_out), jnp.bfloat16),
        in_specs=[
            pl.BlockSpec(memory_space=pltpu.VMEM),
            pl.BlockSpec(memory_space=pltpu.VMEM),
            pl.BlockSpec(memory_space=pl.ANY),
        ],
        out_specs=pl.BlockSpec(memory_space=pltpu.VMEM),
        scratch_shapes=[
            pltpu.VMEM((2, S, C), jnp.float32),
            pltpu.VMEM((N_DEV - 1, S, H), jnp.bfloat16),
            pltpu.VMEM((N_DEV - 1, S, H), jnp.bfloat16),
            pltpu.SemaphoreType.DMA((2, N_DEV - 1)),
            pltpu.SemaphoreType.DMA((2, N_DEV - 1)),
            pltpu.SemaphoreType.DMA((2, N_DEV - 1)),
            pltpu.SemaphoreType.DMA((2, N_DEV - 1)),
            pltpu.SemaphoreType.DMA((2,)),
        ],
        compiler_params=pltpu.CompilerParams(
            collective_id=0,
            vmem_limit_bytes=100 * 1024 * 1024,
        ),
    )(k, Wp, x)
    return out


# device time: 181295 ns/iter; 1.9045x vs baseline; 1.0850x over previous
import jax
import jax.numpy as jnp
from jax import lax
from jax.experimental import pallas as pl
from jax.experimental.pallas import tpu as pltpu

N_DEV = 4
K_TAPS = 4


def kernel(x, k, Wp):
    B, S, C = x.shape
    C_out = Wp.shape[1]
    H = C_out // 2
    S2 = S // 2
    f32 = jnp.float32
    bf16 = jnp.bfloat16

    wp_bf = Wp.astype(bf16)

    def body(k_ref, wp_ref, x_ref, out_ref,
             xbuf, comm_cw, comm_ccw,
             rs_send, rs_recv, ag_send, ag_recv, ld_sems):
        my = lax.axis_index("i")
        left = lax.rem(my + N_DEV - 1, N_DEV)
        right = lax.rem(my + 1, N_DEV)
        order = [my, left, right, lax.rem(my + 2, N_DEV)]

        barrier = pltpu.get_barrier_semaphore()
        for nbr in (left, right):
            pl.semaphore_signal(
                barrier, inc=1,
                device_id=(nbr,), device_id_type=pl.DeviceIdType.MESH,
            )

        kv = k_ref[...].astype(bf16)

        def start_load(i):
            cp = pltpu.make_async_copy(
                x_ref.at[pl.ds(order[i], 1)],
                xbuf.at[pl.ds(i % 2, 1)],
                ld_sems.at[i % 2],
            )
            cp.start()
            return cp

        loads = {0: start_load(0), 1: start_load(1)}

        def compute(i):
            loads[i].wait()
            xa = xbuf[i % 2].astype(bf16)
            pad = jnp.concatenate(
                [jnp.zeros((K_TAPS - 1, C), bf16), xa], axis=0
            )
            acc = pad[0:S] * kv[0]
            for t in range(1, K_TAPS):
                acc = acc + pad[t:t + S] * kv[t]
            a = acc * jax.nn.sigmoid(acc)
            res = jnp.dot(
                a, wp_ref[...], preferred_element_type=f32
            ).astype(bf16)
            out_ref[pl.ds(order[i], 1)] = res[None]

        def rs_copy(dir_idx, t, c_send, comm, nbr):
            return pltpu.make_async_remote_copy(
                src_ref=out_ref.at[pl.ds(c_send, 1), :,
                                   pl.ds(dir_idx * H, H)],
                dst_ref=comm.at[pl.ds(t, 1)],
                send_sem=rs_send.at[dir_idx, t],
                recv_sem=rs_recv.at[dir_idx, t],
                device_id=(nbr,),
                device_id_type=pl.DeviceIdType.MESH,
            )

        def rs_add(dir_idx, t, c_recv, comm):
            sl = (pl.ds(c_recv, 1), slice(None), pl.ds(dir_idx * H, H))
            out_ref[sl] = out_ref[sl] + comm[pl.ds(t, 1)]

        def ag_copy(dir_idx, t, sub, c_send, nbr):
            sl = (pl.ds(c_send, 1), pl.ds(sub * S2, S2),
                  pl.ds(dir_idx * H, H))
            return pltpu.make_async_remote_copy(
                src_ref=out_ref.at[sl],
                dst_ref=out_ref.at[sl],
                send_sem=ag_send.at[dir_idx, t, sub],
                recv_sem=ag_recv.at[dir_idx, t, sub],
                device_id=(nbr,),
                device_id_type=pl.DeviceIdType.MESH,
            )

        compute(0)
        pl.semaphore_wait(barrier, 2)
        cw0 = rs_copy(0, 0, order[0], comm_cw, right)
        ccw0 = rs_copy(1, 0, order[0], comm_ccw, left)
        cw0.start()
        ccw0.start()
        loads[2] = start_load(2)
        compute(1)
        loads[3] = start_load(3)
        compute(2)
        cw0.wait()
        rs_add(0, 0, order[1], comm_cw)
        cw1 = rs_copy(0, 1, order[1], comm_cw, right)
        cw1.start()
        ccw0.wait()
        rs_add(1, 0, order[2], comm_ccw)
        ccw1 = rs_copy(1, 1, order[2], comm_ccw, left)
        ccw1.start()
        compute(3)
        cw1.wait()
        rs_add(0, 1, order[3], comm_cw)
        cw2 = rs_copy(0, 2, order[3], comm_cw, right)
        cw2.start()
        ccw1.wait()
        rs_add(1, 1, order[3], comm_ccw)
        ccw2 = rs_copy(1, 2, order[3], comm_ccw, left)
        ccw2.start()
        cw2.wait()
        rs_add(0, 2, order[2], comm_cw)
        ccw2.wait()
        rs_add(1, 2, order[1], comm_ccw)

        ag_cw_chunks = [order[2], order[0], order[1]]
        ag_ccw_chunks = [order[1], order[0], order[2]]
        ag = {}
        for sub in range(2):
            ag[(0, 0, sub)] = ag_copy(0, 0, sub, ag_cw_chunks[0], right)
            ag[(1, 0, sub)] = ag_copy(1, 0, sub, ag_ccw_chunks[0], left)
            ag[(0, 0, sub)].start()
            ag[(1, 0, sub)].start()
        for t in range(1, N_DEV - 1):
            for sub in range(2):
                ag[(0, t - 1, sub)].wait()
                ag[(0, t, sub)] = ag_copy(0, t, sub, ag_cw_chunks[t], right)
                ag[(0, t, sub)].start()
                ag[(1, t - 1, sub)].wait()
                ag[(1, t, sub)] = ag_copy(1, t, sub, ag_ccw_chunks[t], left)
                ag[(1, t, sub)].start()
        for sub in range(2):
            ag[(0, 2, sub)].wait()
            ag[(1, 2, sub)].wait()

    out = pl.pallas_call(
        body,
        out_shape=jax.ShapeDtypeStruct((B, S, C_out), bf16),
        in_specs=[
            pl.BlockSpec(memory_space=pltpu.VMEM),
            pl.BlockSpec(memory_space=pltpu.VMEM),
            pl.BlockSpec(memory_space=pl.ANY),
        ],
        out_specs=pl.BlockSpec(memory_space=pltpu.VMEM),
        scratch_shapes=[
            pltpu.VMEM((2, S, C), f32),
            pltpu.VMEM((N_DEV - 1, S, H), bf16),
            pltpu.VMEM((N_DEV - 1, S, H), bf16),
            pltpu.SemaphoreType.DMA((2, N_DEV - 1)),
            pltpu.SemaphoreType.DMA((2, N_DEV - 1)),
            pltpu.SemaphoreType.DMA((2, N_DEV - 1, 2)),
            pltpu.SemaphoreType.DMA((2, N_DEV - 1, 2)),
            pltpu.SemaphoreType.DMA((2,)),
        ],
        compiler_params=pltpu.CompilerParams(
            collective_id=0,
            vmem_limit_bytes=100 * 1024 * 1024,
        ),
    )(k, wp_bf, x)
    return out


# device time: 175934 ns/iter; 1.9625x vs baseline; 1.0305x over previous
import jax
import jax.numpy as jnp
from jax import lax
from jax.experimental import pallas as pl
from jax.experimental.pallas import tpu as pltpu

N_DEV = 4
K_TAPS = 4


def kernel(x, k, Wp):
    B, S, C = x.shape
    C_out = Wp.shape[1]
    H = C_out // 2
    S2 = S // 2
    f32 = jnp.float32
    bf16 = jnp.bfloat16

    wp_bf = Wp.astype(bf16)

    def body(k_ref, wp_ref, x_ref, out_ref,
             xbuf, comm_cw, comm_ccw,
             rs_send, rs_recv, ag_send, ag_recv, ld_sems):
        my = lax.axis_index("i")
        left = lax.rem(my + N_DEV - 1, N_DEV)
        right = lax.rem(my + 1, N_DEV)
        order = [my, left, right, lax.rem(my + 2, N_DEV)]

        barrier = pltpu.get_barrier_semaphore()
        for nbr in (left, right):
            pl.semaphore_signal(
                barrier, inc=1,
                device_id=(nbr,), device_id_type=pl.DeviceIdType.MESH,
            )

        kv = k_ref[...].astype(bf16)

        def start_load(i):
            cp = pltpu.make_async_copy(
                x_ref.at[pl.ds(order[i], 1)],
                xbuf.at[pl.ds(i % 2, 1)],
                ld_sems.at[i % 2],
            )
            cp.start()
            return cp

        loads = {0: start_load(0), 1: start_load(1)}

        def compute(i):
            loads[i].wait()
            xa = xbuf[i % 2].astype(bf16)
            pad = jnp.concatenate(
                [jnp.zeros((K_TAPS - 1, C), bf16), xa], axis=0
            )
            acc = pad[0:S] * kv[0]
            for t in range(1, K_TAPS):
                acc = acc + pad[t:t + S] * kv[t]
            a = acc * jax.nn.sigmoid(acc)
            res = jnp.dot(
                a, wp_ref[...], preferred_element_type=f32
            ).astype(bf16)
            out_ref[pl.ds(order[i], 1)] = res[None]

        def rs_copy(dir_idx, t, sub, c_send, comm, nbr):
            sl = (pl.ds(c_send, 1), pl.ds(sub * S2, S2),
                  pl.ds(dir_idx * H, H))
            return pltpu.make_async_remote_copy(
                src_ref=out_ref.at[sl],
                dst_ref=comm.at[pl.ds(t, 1), pl.ds(sub * S2, S2)],
                send_sem=rs_send.at[dir_idx, t, sub],
                recv_sem=rs_recv.at[dir_idx, t, sub],
                device_id=(nbr,),
                device_id_type=pl.DeviceIdType.MESH,
            )

        def rs_add(dir_idx, t, sub, c_recv, comm):
            sl = (pl.ds(c_recv, 1), pl.ds(sub * S2, S2),
                  pl.ds(dir_idx * H, H))
            out_ref[sl] = (
                out_ref[sl]
                + comm[pl.ds(t, 1), pl.ds(sub * S2, S2)]
            )

        def ag_copy(dir_idx, t, sub, c_send, nbr):
            sl = (pl.ds(c_send, 1), pl.ds(sub * S2, S2),
                  pl.ds(dir_idx * H, H))
            return pltpu.make_async_remote_copy(
                src_ref=out_ref.at[sl],
                dst_ref=out_ref.at[sl],
                send_sem=ag_send.at[dir_idx, t, sub],
                recv_sem=ag_recv.at[dir_idx, t, sub],
                device_id=(nbr,),
                device_id_type=pl.DeviceIdType.MESH,
            )

        cw_send = [order[0], order[1], order[3]]
        cw_recv = [order[1], order[3], order[2]]
        ccw_send = [order[0], order[2], order[3]]
        ccw_recv = [order[2], order[3], order[1]]

        rs = {}

        def rs_start(dir_idx, t, sub):
            comm, nbr = ((comm_cw, right) if dir_idx == 0
                         else (comm_ccw, left))
            c = cw_send[t] if dir_idx == 0 else ccw_send[t]
            rs[(dir_idx, t, sub)] = rs_copy(dir_idx, t, sub, c, comm, nbr)
            rs[(dir_idx, t, sub)].start()

        def rs_finish(dir_idx, t, sub):
            comm = comm_cw if dir_idx == 0 else comm_ccw
            c = cw_recv[t] if dir_idx == 0 else ccw_recv[t]
            rs[(dir_idx, t, sub)].wait()
            rs_add(dir_idx, t, sub, c, comm)

        compute(0)
        pl.semaphore_wait(barrier, 2)
        for sub in range(2):
            rs_start(0, 0, sub)
            rs_start(1, 0, sub)
        loads[2] = start_load(2)
        compute(1)
        rs_finish(0, 0, 0)
        rs_start(0, 1, 0)
        loads[3] = start_load(3)
        compute(2)
        rs_finish(0, 0, 1)
        rs_start(0, 1, 1)
        rs_finish(1, 0, 0)
        rs_start(1, 1, 0)
        rs_finish(1, 0, 1)
        rs_start(1, 1, 1)
        compute(3)
        rs_finish(0, 1, 0)
        rs_start(0, 2, 0)
        rs_finish(0, 1, 1)
        rs_start(0, 2, 1)
        rs_finish(1, 1, 0)
        rs_start(1, 2, 0)
        rs_finish(1, 1, 1)
        rs_start(1, 2, 1)

        ag_cw_chunks = [order[2], order[0], order[1]]
        ag_ccw_chunks = [order[1], order[0], order[2]]
        ag = {}
        for sub in range(2):
            rs_finish(0, 2, sub)
            ag[(0, 0, sub)] = ag_copy(0, 0, sub, ag_cw_chunks[0], right)
            ag[(0, 0, sub)].start()
            rs_finish(1, 2, sub)
            ag[(1, 0, sub)] = ag_copy(1, 0, sub, ag_ccw_chunks[0], left)
            ag[(1, 0, sub)].start()
        for t in range(1, N_DEV - 1):
            for sub in range(2):
                ag[(0, t - 1, sub)].wait()
                ag[(0, t, sub)] = ag_copy(0, t, sub, ag_cw_chunks[t], right)
                ag[(0, t, sub)].start()
                ag[(1, t - 1, sub)].wait()
                ag[(1, t, sub)] = ag_copy(1, t, sub, ag_ccw_chunks[t], left)
                ag[(1, t, sub)].start()
        for sub in range(2):
            ag[(0, 2, sub)].wait()
            ag[(1, 2, sub)].wait()

    out = pl.pallas_call(
        body,
        out_shape=jax.ShapeDtypeStruct((B, S, C_out), bf16),
        in_specs=[
            pl.BlockSpec(memory_space=pltpu.VMEM),
            pl.BlockSpec(memory_space=pltpu.VMEM),
            pl.BlockSpec(memory_space=pl.ANY),
        ],
        out_specs=pl.BlockSpec(memory_space=pltpu.VMEM),
        scratch_shapes=[
            pltpu.VMEM((2, S, C), f32),
            pltpu.VMEM((N_DEV - 1, S, H), bf16),
            pltpu.VMEM((N_DEV - 1, S, H), bf16),
            pltpu.SemaphoreType.DMA((2, N_DEV - 1, 2)),
            pltpu.SemaphoreType.DMA((2, N_DEV - 1, 2)),
            pltpu.SemaphoreType.DMA((2, N_DEV - 1, 2)),
            pltpu.SemaphoreType.DMA((2, N_DEV - 1, 2)),
            pltpu.SemaphoreType.DMA((2,)),
        ],
        compiler_params=pltpu.CompilerParams(
            collective_id=0,
            vmem_limit_bytes=100 * 1024 * 1024,
        ),
    )(k, wp_bf, x)
    return out


# device time: 175816 ns/iter; 1.9638x vs baseline; 1.0007x over previous
import jax
import jax.numpy as jnp
from jax import lax
from jax.experimental import pallas as pl
from jax.experimental.pallas import tpu as pltpu

N_DEV = 4
K_TAPS = 4


def kernel(x, k, Wp):
    B, S, C = x.shape
    C_out = Wp.shape[1]
    H = C_out // 2
    S2 = S // 2
    f32 = jnp.float32
    bf16 = jnp.bfloat16

    wp_bf = Wp.astype(bf16)

    def body(k_ref, wp_ref, x_ref, out_ref,
             xbuf, comm_cw, comm_ccw,
             rs_send, rs_recv, ag_send, ag_recv, ld_sems):
        my = lax.axis_index("i")
        left = lax.rem(my + N_DEV - 1, N_DEV)
        right = lax.rem(my + 1, N_DEV)
        order = [my, left, right, lax.rem(my + 2, N_DEV)]

        barrier = pltpu.get_barrier_semaphore()
        for nbr in (left, right):
            pl.semaphore_signal(
                barrier, inc=1,
                device_id=(nbr,), device_id_type=pl.DeviceIdType.MESH,
            )

        kv = k_ref[...].astype(bf16)

        def start_load(i):
            cp = pltpu.make_async_copy(
                x_ref.at[pl.ds(order[i], 1)],
                xbuf.at[pl.ds(i % 2, 1)],
                ld_sems.at[i % 2],
            )
            cp.start()
            return cp

        loads = {0: start_load(0), 1: start_load(1)}

        def compute_half(i, half):
            if half == 0:
                loads[i].wait()
                pad = jnp.concatenate(
                    [jnp.zeros((K_TAPS - 1, C), bf16),
                     xbuf[i % 2, 0:S2].astype(bf16)], axis=0
                )
            else:
                pad = xbuf[i % 2, S2 - (K_TAPS - 1):S].astype(bf16)
            acc = pad[0:S2] * kv[0]
            for t in range(1, K_TAPS):
                acc = acc + pad[t:t + S2] * kv[t]
            a = acc * jax.nn.sigmoid(acc)
            res = jnp.dot(
                a, wp_ref[...], preferred_element_type=f32
            ).astype(bf16)
            out_ref[pl.ds(order[i], 1), pl.ds(half * S2, S2)] = res[None]

        def rs_copy(dir_idx, t, sub, c_send, comm, nbr):
            sl = (pl.ds(c_send, 1), pl.ds(sub * S2, S2),
                  pl.ds(dir_idx * H, H))
            return pltpu.make_async_remote_copy(
                src_ref=out_ref.at[sl],
                dst_ref=comm.at[pl.ds(t, 1), pl.ds(sub * S2, S2)],
                send_sem=rs_send.at[dir_idx, t, sub],
                recv_sem=rs_recv.at[dir_idx, t, sub],
                device_id=(nbr,),
                device_id_type=pl.DeviceIdType.MESH,
            )

        def rs_add(dir_idx, t, sub, c_recv, comm):
            sl = (pl.ds(c_recv, 1), pl.ds(sub * S2, S2),
                  pl.ds(dir_idx * H, H))
            out_ref[sl] = (
                out_ref[sl]
                + comm[pl.ds(t, 1), pl.ds(sub * S2, S2)]
            )

        def ag_copy(dir_idx, t, sub, c_send, nbr):
            sl = (pl.ds(c_send, 1), pl.ds(sub * S2, S2),
                  pl.ds(dir_idx * H, H))
            return pltpu.make_async_remote_copy(
                src_ref=out_ref.at[sl],
                dst_ref=out_ref.at[sl],
                send_sem=ag_send.at[dir_idx, t, sub],
                recv_sem=ag_recv.at[dir_idx, t, sub],
                device_id=(nbr,),
                device_id_type=pl.DeviceIdType.MESH,
            )

        cw_send = [order[0], order[1], order[3]]
        cw_recv = [order[1], order[3], order[2]]
        ccw_send = [order[0], order[2], order[3]]
        ccw_recv = [order[2], order[3], order[1]]

        rs = {}

        def rs_start(dir_idx, t, sub):
            comm, nbr = ((comm_cw, right) if dir_idx == 0
                         else (comm_ccw, left))
            c = cw_send[t] if dir_idx == 0 else ccw_send[t]
            rs[(dir_idx, t, sub)] = rs_copy(dir_idx, t, sub, c, comm, nbr)
            rs[(dir_idx, t, sub)].start()

        def rs_finish(dir_idx, t, sub):
            comm = comm_cw if dir_idx == 0 else comm_ccw
            c = cw_recv[t] if dir_idx == 0 else ccw_recv[t]
            rs[(dir_idx, t, sub)].wait()
            rs_add(dir_idx, t, sub, c, comm)

        compute_half(0, 0)
        pl.semaphore_wait(barrier, 2)
        rs_start(0, 0, 0)
        rs_start(1, 0, 0)
        compute_half(0, 1)
        rs_start(0, 0, 1)
        rs_start(1, 0, 1)
        loads[2] = start_load(2)
        compute_half(1, 0)
        rs_finish(0, 0, 0)
        rs_start(0, 1, 0)
        compute_half(1, 1)
        rs_finish(0, 0, 1)
        rs_start(0, 1, 1)
        loads[3] = start_load(3)
        compute_half(2, 0)
        rs_finish(1, 0, 0)
        rs_start(1, 1, 0)
        compute_half(2, 1)
        rs_finish(1, 0, 1)
        rs_start(1, 1, 1)
        compute_half(3, 0)
        rs_finish(0, 1, 0)
        rs_start(0, 2, 0)
        rs_finish(1, 1, 0)
        rs_start(1, 2, 0)
        compute_half(3, 1)
        rs_finish(0, 1, 1)
        rs_start(0, 2, 1)
        rs_finish(1, 1, 1)
        rs_start(1, 2, 1)

        ag_cw_chunks = [order[2], order[0], order[1]]
        ag_ccw_chunks = [order[1], order[0], order[2]]
        ag = {}
        for sub in range(2):
            rs_finish(0, 2, sub)
            ag[(0, 0, sub)] = ag_copy(0, 0, sub, ag_cw_chunks[0], right)
            ag[(0, 0, sub)].start()
            rs_finish(1, 2, sub)
            ag[(1, 0, sub)] = ag_copy(1, 0, sub, ag_ccw_chunks[0], left)
            ag[(1, 0, sub)].start()
        for t in range(1, N_DEV - 1):
            for sub in range(2):
                ag[(0, t - 1, sub)].wait()
                ag[(0, t, sub)] = ag_copy(0, t, sub, ag_cw_chunks[t], right)
                ag[(0, t, sub)].start()
                ag[(1, t - 1, sub)].wait()
                ag[(1, t, sub)] = ag_copy(1, t, sub, ag_ccw_chunks[t], left)
                ag[(1, t, sub)].start()
        for sub in range(2):
            ag[(0, 2, sub)].wait()
            ag[(1, 2, sub)].wait()

    out = pl.pallas_call(
        body,
        out_shape=jax.ShapeDtypeStruct((B, S, C_out), bf16),
        in_specs=[
            pl.BlockSpec(memory_space=pltpu.VMEM),
            pl.BlockSpec(memory_space=pltpu.VMEM),
            pl.BlockSpec(memory_space=pl.ANY),
        ],
        out_specs=pl.BlockSpec(memory_space=pltpu.VMEM),
        scratch_shapes=[
            pltpu.VMEM((2, S, C), f32),
            pltpu.VMEM((N_DEV - 1, S, H), bf16),
            pltpu.VMEM((N_DEV - 1, S, H), bf16),
            pltpu.SemaphoreType.DMA((2, N_DEV - 1, 2)),
            pltpu.SemaphoreType.DMA((2, N_DEV - 1, 2)),
            pltpu.SemaphoreType.DMA((2, N_DEV - 1, 2)),
            pltpu.SemaphoreType.DMA((2, N_DEV - 1, 2)),
            pltpu.SemaphoreType.DMA((2,)),
        ],
        compiler_params=pltpu.CompilerParams(
            collective_id=0,
            vmem_limit_bytes=100 * 1024 * 1024,
        ),
    )(k, wp_bf, x)
    return out
